# baseline (speedup 1.0000x reference)
import functools

import jax
import jax.numpy as jnp
from jax import lax
from jax.experimental import pallas as pl
from jax.experimental.pallas import tpu as pltpu
from jax.experimental.pallas import tpu_sc as plsc

NC = 2
NS = 16
L = 16
NW = NC * NS

A = 8
E = 128
NSAMP = 2048
NZ = 512
NX = 256
ZW = NZ // NW
ZH = ZW // 2
HALF = ZH * NX
EB = 2
NBLK = E // EB
ATBL = EB * NSAMP
BTBL = A * ATBL


def _das_sc(rf3, dtx3, drx3, apod3, kvec, acst):
    mesh = plsc.VectorSubcoreMesh(
        core_axis_name="c", subcore_axis_name="s", num_cores=NC, num_subcores=NS
    )

    @functools.partial(
        pl.kernel,
        out_type=jax.ShapeDtypeStruct((A, NZ, NX), jnp.float32),
        mesh=mesh,
        compiler_params=pltpu.CompilerParams(needs_layout_passes=False),
        scratch_types=[
            pltpu.VMEM((BTBL,), jnp.int32),
            pltpu.VMEM((BTBL,), jnp.int32),
            pltpu.VMEM((2, EB, ZH, NX), jnp.float32),
            pltpu.VMEM((2, EB, ZH, NX), jnp.float32),
            pltpu.VMEM((A, ZH, NX), jnp.float32),
            pltpu.VMEM((A, ZH, NX), jnp.float32),
            pltpu.VMEM((L,), jnp.float32),
            pltpu.VMEM((A, L), jnp.float32),
            pltpu.SemaphoreType.DMA,
            pltpu.SemaphoreType.DMA,
            pltpu.SemaphoreType.DMA,
            pltpu.SemaphoreType.DMA,
            pltpu.SemaphoreType.DMA,
            pltpu.SemaphoreType.DMA,
        ],
    )
    def k(rf_hbm, dtx_hbm, drx_hbm, apod_hbm, kvec_hbm, acst_hbm, out_hbm,
          rf_v0, rf_v1, drx_v, apod_v, dtx_v, acc_v, k_v, ac_v,
          rf_s0, rf_s1, drx_s0, drx_s1, ap_s0, ap_s1):
        wid = lax.axis_index("s") * NC + lax.axis_index("c")
        rf_bufs = (rf_v0, rf_v1)
        rf_sems = (rf_s0, rf_s1)
        drx_sems = (drx_s0, drx_s1)
        ap_sems = (ap_s0, ap_s1)

        pltpu.sync_copy(kvec_hbm, k_v)
        pltpu.sync_copy(acst_hbm, ac_v)
        kreg = k_v[...]
        zeros = jnp.zeros((L,), jnp.float32)

        def start_block(eb, b):
            for a in range(A):
                for j in range(EB):
                    pltpu.async_copy(
                        rf_hbm.at[a, eb * EB + j, :],
                        rf_bufs[b].at[pl.ds(a * ATBL + j * NSAMP, NSAMP)],
                        rf_sems[b])
            pltpu.async_copy(
                drx_hbm.at[pl.ds(eb * EB, EB), pl.ds(zb, ZH), :],
                drx_v.at[b], drx_sems[b])
            pltpu.async_copy(
                apod_hbm.at[pl.ds(eb * EB, EB), pl.ds(zb, ZH), :],
                apod_v.at[b], ap_sems[b])

        def wait_block(eb, b):
            for a in range(A):
                for j in range(EB):
                    pltpu.make_async_copy(
                        rf_hbm.at[a, eb * EB + j, :],
                        rf_bufs[b].at[pl.ds(a * ATBL + j * NSAMP, NSAMP)],
                        rf_sems[b]).wait()
            pltpu.make_async_copy(
                drx_hbm.at[pl.ds(eb * EB, EB), pl.ds(zb, ZH), :],
                drx_v.at[b], drx_sems[b]).wait()
            pltpu.make_async_copy(
                apod_hbm.at[pl.ds(eb * EB, EB), pl.ds(zb, ZH), :],
                apod_v.at[b], ap_sems[b]).wait()

        for half in range(2):
            zb = wid * ZW + half * ZH
            pltpu.sync_copy(dtx_hbm.at[:, pl.ds(zb, ZH), :], dtx_v)

            @plsc.parallel_loop(0, HALF // L)
            def _pre(i):
                zr = lax.shift_right_logical(i, 4)
                co = lax.shift_left(jnp.bitwise_and(i, 15), 4)
                px = pl.ds(co, L)
                for a in range(A):
                    dtx_v[a, zr, px] = dtx_v[a, zr, px] * kreg + ac_v[a, :]
                    acc_v[a, zr, px] = zeros

            start_block(0, 0)
            start_block(1, 1)

            def bb_body(bb, _):
                for b in (0, 1):
                    eb = bb * 2 + b
                    wait_block(eb, b)

                    @plsc.parallel_loop(0, HALF // L)
                    def _px(i):
                        zr = lax.shift_right_logical(i, 4)
                        co = lax.shift_left(jnp.bitwise_and(i, 15), 4)
                        px = pl.ds(co, L)
                        rf_b = rf_bufs[b]
                        drxks = []
                        aps = []
                        for j in range(EB):
                            if j == 0:
                                drxks.append(drx_v[b, j, zr, px] * kreg)
                            else:
                                drxks.append(drx_v[b, j, zr, px] * kreg
                                             + jnp.float32(j * NSAMP))
                            aps.append(apod_v[b, j, zr, px])
                        for a in range(A):
                            dtx_a = dtx_v[a, zr, px]
                            contrib = None
                            for j in range(EB):
                                d = dtx_a + drxks[j]
                                i0 = d.astype(jnp.int32)
                                w = d - i0.astype(jnp.float32)
                                g = plsc.load_gather(rf_b, [i0])
                                vb = plsc.bitcast(g, jnp.bfloat16)
                                v0, dv = plsc.unpack(
                                    vb, format=plsc.PackFormat.INTERLEAVED)
                                t = (v0 + w * dv) * aps[j]
                                contrib = t if contrib is None else contrib + t
                            plsc.addupdate(acc_v.at[a, zr, px], contrib)
                        return None

                    @pl.when(eb + 2 < NBLK)
                    def _():
                        start_block(eb + 2, b)
                return 0

            lax.fori_loop(0, NBLK // 2, bb_body, 0)
            pltpu.sync_copy(acc_v, out_hbm.at[:, pl.ds(zb, ZH), :])

    return k(rf3, dtx3, drx3, apod3, kvec, acst)


def kernel(rf, t0, d_tx, d_rx, fs, c0, apod):
    n_angles, n_elements, n_samp = rf.shape
    kscale = (fs / c0).astype(jnp.float32)
    offs = (jnp.arange(n_angles, dtype=jnp.float32) * (EB * n_samp)
            - t0.astype(jnp.float32) * fs)
    acst = jnp.broadcast_to(offs[:, None], (n_angles, L))
    kvec = jnp.full((L,), kscale, jnp.float32)
    dv = jnp.concatenate(
        [rf[:, :, 1:], jnp.zeros_like(rf[:, :, :1])], axis=2) - rf
    lo = lax.bitcast_convert_type(rf.astype(jnp.bfloat16), jnp.uint16)
    hi = lax.bitcast_convert_type(dv.astype(jnp.bfloat16), jnp.uint16)
    packed = lax.bitcast_convert_type(
        lo.astype(jnp.uint32) | (hi.astype(jnp.uint32) << 16), jnp.int32)
    return _das_sc(packed, d_tx, d_rx, apod, kvec, acst)


# speedup vs baseline: 1.2260x; 1.2260x over previous
import functools

import jax
import jax.numpy as jnp
from jax import lax
from jax.experimental import pallas as pl
from jax.experimental.pallas import tpu as pltpu
from jax.experimental.pallas import tpu_sc as plsc

NC = 2
NS = 16
L = 16
NW = NC * NS

A = 8
E = 128
NSAMP = 2048
NZ = 512
NX = 256
ZW = NZ // NW
ZH = ZW // 2
HALF = ZH * NX
EB = 2
NBLK = E // EB
ATBL = EB * NSAMP
BTBL = A * ATBL


def _das_sc(rf3, dtx3, drx3, apod3, kvec, acst):
    mesh = plsc.VectorSubcoreMesh(
        core_axis_name="c", subcore_axis_name="s", num_cores=NC, num_subcores=NS
    )

    @functools.partial(
        pl.kernel,
        out_type=jax.ShapeDtypeStruct((A, NZ, NX), jnp.float32),
        mesh=mesh,
        compiler_params=pltpu.CompilerParams(needs_layout_passes=False),
        scratch_types=[
            pltpu.VMEM((BTBL + L,), jnp.float32),
            pltpu.VMEM((BTBL + L,), jnp.float32),
            pltpu.VMEM((2, EB, ZH, NX), jnp.float32),
            pltpu.VMEM((2, EB, ZH, NX), jnp.float32),
            pltpu.VMEM((A, ZH, NX), jnp.float32),
            pltpu.VMEM((A, ZH, NX), jnp.float32),
            pltpu.VMEM((L,), jnp.float32),
            pltpu.VMEM((A, L), jnp.float32),
            pltpu.SemaphoreType.DMA,
            pltpu.SemaphoreType.DMA,
            pltpu.SemaphoreType.DMA,
            pltpu.SemaphoreType.DMA,
            pltpu.SemaphoreType.DMA,
            pltpu.SemaphoreType.DMA,
        ],
    )
    def k(rf_hbm, dtx_hbm, drx_hbm, apod_hbm, kvec_hbm, acst_hbm, out_hbm,
          rf_v0, rf_v1, drx_v, apod_v, dtx_v, acc_v, k_v, ac_v,
          rf_s0, rf_s1, drx_s0, drx_s1, ap_s0, ap_s1):
        wid = lax.axis_index("s") * NC + lax.axis_index("c")
        rf_bufs = (rf_v0, rf_v1)
        rf_sems = (rf_s0, rf_s1)
        drx_sems = (drx_s0, drx_s1)
        ap_sems = (ap_s0, ap_s1)

        pltpu.sync_copy(kvec_hbm, k_v)
        pltpu.sync_copy(acst_hbm, ac_v)
        kreg = k_v[...]
        zeros = jnp.zeros((L,), jnp.float32)
        for b in (0, 1):
            rf_bufs[b][pl.ds(BTBL, L)] = zeros

        def start_block(eb, b):
            for a in range(A):
                for j in range(EB):
                    pltpu.async_copy(
                        rf_hbm.at[a, eb * EB + j, :],
                        rf_bufs[b].at[pl.ds(a * ATBL + j * NSAMP, NSAMP)],
                        rf_sems[b])
            pltpu.async_copy(
                drx_hbm.at[pl.ds(eb * EB, EB), pl.ds(zb, ZH), :],
                drx_v.at[b], drx_sems[b])
            pltpu.async_copy(
                apod_hbm.at[pl.ds(eb * EB, EB), pl.ds(zb, ZH), :],
                apod_v.at[b], ap_sems[b])

        def wait_block(eb, b):
            for a in range(A):
                for j in range(EB):
                    pltpu.make_async_copy(
                        rf_hbm.at[a, eb * EB + j, :],
                        rf_bufs[b].at[pl.ds(a * ATBL + j * NSAMP, NSAMP)],
                        rf_sems[b]).wait()
            pltpu.make_async_copy(
                drx_hbm.at[pl.ds(eb * EB, EB), pl.ds(zb, ZH), :],
                drx_v.at[b], drx_sems[b]).wait()
            pltpu.make_async_copy(
                apod_hbm.at[pl.ds(eb * EB, EB), pl.ds(zb, ZH), :],
                apod_v.at[b], ap_sems[b]).wait()

        for half in range(2):
            zb = wid * ZW + half * ZH
            pltpu.sync_copy(dtx_hbm.at[:, pl.ds(zb, ZH), :], dtx_v)

            @plsc.parallel_loop(0, HALF // L)
            def _pre(i):
                zr = lax.shift_right_logical(i, 4)
                co = lax.shift_left(jnp.bitwise_and(i, 15), 4)
                px = pl.ds(co, L)
                for a in range(A):
                    dtx_v[a, zr, px] = dtx_v[a, zr, px] * kreg + ac_v[a, :]
                    acc_v[a, zr, px] = zeros

            start_block(0, 0)
            start_block(1, 1)

            def bb_body(bb, _):
                for b in (0, 1):
                    eb = bb * 2 + b
                    wait_block(eb, b)

                    @plsc.parallel_loop(0, HALF // L)
                    def _px(i):
                        zr = lax.shift_right_logical(i, 4)
                        co = lax.shift_left(jnp.bitwise_and(i, 15), 4)
                        px = pl.ds(co, L)
                        rf_b = rf_bufs[b]
                        drxks = []
                        aps = []
                        for j in range(EB):
                            if j == 0:
                                drxks.append(drx_v[b, j, zr, px] * kreg)
                            else:
                                drxks.append(drx_v[b, j, zr, px] * kreg
                                             + jnp.float32(j * NSAMP))
                            aps.append(apod_v[b, j, zr, px])
                        for a in range(A):
                            dtx_a = dtx_v[a, zr, px]
                            contrib = None
                            for j in range(EB):
                                d = dtx_a + drxks[j]
                                i0 = d.astype(jnp.int32)
                                w = d - i0.astype(jnp.float32)
                                v0 = plsc.load_gather(rf_b, [i0])
                                v1 = plsc.load_gather(rf_b, [i0 + 1])
                                t = (v0 + w * (v1 - v0)) * aps[j]
                                contrib = t if contrib is None else contrib + t
                            plsc.addupdate(acc_v.at[a, zr, px], contrib)
                        return None

                    @pl.when(eb + 2 < NBLK)
                    def _():
                        start_block(eb + 2, b)
                return 0

            lax.fori_loop(0, NBLK // 2, bb_body, 0)
            pltpu.sync_copy(acc_v, out_hbm.at[:, pl.ds(zb, ZH), :])

    return k(rf3, dtx3, drx3, apod3, kvec, acst)


def kernel(rf, t0, d_tx, d_rx, fs, c0, apod):
    n_angles, n_elements, n_samp = rf.shape
    kscale = (fs / c0).astype(jnp.float32)
    offs = (jnp.arange(n_angles, dtype=jnp.float32) * (EB * n_samp)
            - t0.astype(jnp.float32) * fs)
    acst = jnp.broadcast_to(offs[:, None], (n_angles, L))
    kvec = jnp.full((L,), kscale, jnp.float32)
    return _das_sc(rf, d_tx, d_rx, apod, kvec, acst)
